# Initial kernel scaffold; baseline (speedup 1.0000x reference)
import jax
import jax.numpy as jnp
from jax.experimental import pallas as pl


def kernel(x, emb_s, emb_t):
    raise NotImplementedError("write your pallas kernel here")


# speedup vs baseline: 3.9809x; 3.9809x over previous
import functools

import jax
import jax.numpy as jnp
from jax import lax
from jax.experimental import pallas as pl
from jax.experimental.pallas import tpu as pltpu
from jax.experimental.pallas import tpu_sc as plsc

_B, _L, _F = 16384, 50, 10
_DS, _DT = 32, 32
_NSYM, _NTIME = 100, 2000
_DO = _F - 2 + _DS + _DT

_NC, _NS, _LANES = 2, 16, 16
_NW = _NC * _NS
_N_TOK = _B * _L
_TPW = _N_TOK // _NW
_T = 256
_NCHUNK = _TPW // _T
_G = _T // _LANES


def _body(x_hbm, embs_hbm, embt_hbm, out_hbm, embs_v, embt_v, xv, ov, sem):
    wid = lax.axis_index("s") * _NC + lax.axis_index("c")
    base = wid * _TPW

    pltpu.async_copy(embs_hbm, embs_v, sem).wait()
    pltpu.async_copy(embt_hbm, embt_v, sem).wait()

    iota = lax.iota(jnp.int32, _LANES)

    def chunk(i, carry):
        t0 = base + i * _T
        pltpu.async_copy(x_hbm.at[pl.ds(t0 * _F, _T * _F)], xv, sem).wait()

        def group(g, carry2):
            tv = iota + g * _LANES
            xb = tv * _F
            ob = tv * _DO
            srow = plsc.load_gather(xv, [xb + (_F - 2)]).astype(jnp.int32) * _DS
            trow = plsc.load_gather(xv, [xb + (_F - 1)]).astype(jnp.int32) * _DT
            for c in range(_F - 2):
                v = plsc.load_gather(xv, [xb + c])
                plsc.store_scatter(ov, [ob + c], v)
            for d in range(_DS):
                v = plsc.load_gather(embs_v, [srow + d])
                plsc.store_scatter(ov, [ob + (_F - 2) + d], v)
            for d in range(_DT):
                v = plsc.load_gather(embt_v, [trow + d])
                plsc.store_scatter(ov, [ob + (_F - 2 + _DS) + d], v)
            return carry2

        lax.fori_loop(0, _G, group, 0)
        pltpu.async_copy(ov, out_hbm.at[pl.ds(t0 * _DO, _T * _DO)], sem).wait()
        return carry

    lax.fori_loop(0, _NCHUNK, chunk, 0)


@jax.jit
def _sc_embed(xf, embs, embt):
    mesh = plsc.VectorSubcoreMesh(core_axis_name="c", subcore_axis_name="s")
    return pl.kernel(
        _body,
        out_type=jax.ShapeDtypeStruct((_N_TOK * _DO,), jnp.float32),
        mesh=mesh,
        compiler_params=pltpu.CompilerParams(needs_layout_passes=False),
        scratch_types=[
            pltpu.VMEM((_NSYM * _DS,), jnp.float32),
            pltpu.VMEM((_NTIME * _DT,), jnp.float32),
            pltpu.VMEM((_T * _F,), jnp.float32),
            pltpu.VMEM((_T * _DO,), jnp.float32),
            pltpu.SemaphoreType.DMA,
        ],
    )(xf, embs, embt)


def kernel(x, emb_s, emb_t):
    xf = x.reshape(_N_TOK * _F)
    out = _sc_embed(xf, emb_s.reshape(-1), emb_t.reshape(-1))
    return out.reshape(_B, _L, _DO)


# speedup vs baseline: 3.9958x; 1.0037x over previous
import jax
import jax.numpy as jnp
from jax import lax
from jax.experimental import pallas as pl
from jax.experimental.pallas import tpu as pltpu
from jax.experimental.pallas import tpu_sc as plsc

_B, _L, _F = 16384, 50, 10
_DS, _DT = 32, 32
_NSYM, _NTIME = 100, 2000
_DO = _F - 2 + _DS + _DT
_DOP = _DO + 1
_DSP, _DTP = _DS + 1, _DT + 1

_NC, _NS, _LANES = 2, 16, 16
_NW = _NC * _NS
_N_TOK = _B * _L
_TPW = _N_TOK // _NW
_T = 256
_NCHUNK = _TPW // _T
_G = _T // _LANES


def _body(x_hbm, embs_hbm, embt_hbm, out_hbm,
          embs_v, embt_v, xv0, xv1, ov0, ov1, sin, sout, stab):
    wid = lax.axis_index("s") * _NC + lax.axis_index("c")
    base = wid * _TPW
    xvs = (xv0, xv1)
    ovs = (ov0, ov1)

    pltpu.async_copy(embs_hbm, embs_v.at[:, pl.ds(0, _DS)], stab).wait()
    pltpu.async_copy(embt_hbm, embt_v.at[:, pl.ds(0, _DT)], stab).wait()

    iota = lax.iota(jnp.int32, _LANES)

    def start_in(i, b):
        t0 = base + i * _T
        return pltpu.async_copy(x_hbm.at[pl.ds(t0 * _F, _T * _F)], xvs[b],
                                sin[b])

    def wait_in(b):
        pltpu.make_async_copy(x_hbm.at[pl.ds(0, _T * _F)], xvs[b],
                              sin[b]).wait()

    def start_out(i, b):
        t0 = base + i * _T
        return pltpu.async_copy(ovs[b].at[:, pl.ds(0, _DO)],
                                out_hbm.at[pl.ds(t0, _T)], sout[b])

    def wait_out(b):
        pltpu.make_async_copy(out_hbm.at[pl.ds(0, _T)],
                              ovs[b].at[:, pl.ds(0, _DO)], sout[b]).wait()

    def compute(b):
        xv, ov = xvs[b], ovs[b]

        def group(g, carry):
            tv = iota + g * _LANES
            xb = tv * _F
            srow = plsc.load_gather(xv, [xb + (_F - 2)]).astype(jnp.int32)
            trow = plsc.load_gather(xv, [xb + (_F - 1)]).astype(jnp.int32)
            for c in range(_F - 2):
                v = plsc.load_gather(xv, [xb + c])
                plsc.store_scatter(ov, [tv, jnp.full((_LANES,), c,
                                                     jnp.int32)], v)
            for d in range(_DS):
                v = plsc.load_gather(embs_v, [srow,
                                              jnp.full((_LANES,), d, jnp.int32)])
                plsc.store_scatter(ov, [tv, jnp.full((_LANES,), _F - 2 + d,
                                                     jnp.int32)], v)
            for d in range(_DT):
                v = plsc.load_gather(embt_v, [trow,
                                              jnp.full((_LANES,), d, jnp.int32)])
                plsc.store_scatter(ov,
                                   [tv, jnp.full((_LANES,), _F - 2 + _DS + d,
                                                 jnp.int32)], v)
            return carry

        lax.fori_loop(0, _G, group, 0)

    start_in(0, 0)
    start_in(1, 1)

    def pair(j, carry):
        i0 = j * 2
        for b in range(2):
            i = i0 + b
            @pl.when(i >= 2)
            def _():
                wait_out(b)
            wait_in(b)
            compute(b)
            start_out(i, b)

            @pl.when(i + 2 < _NCHUNK)
            def _():
                start_in(i + 2, b)
        return carry

    lax.fori_loop(0, _NCHUNK // 2, pair, 0)
    wait_out(0)
    wait_out(1)


@jax.jit
def _sc_embed(xf, embs, embt):
    mesh = plsc.VectorSubcoreMesh(core_axis_name="c", subcore_axis_name="s")
    return pl.kernel(
        _body,
        out_type=jax.ShapeDtypeStruct((_N_TOK, _DO), jnp.float32),
        mesh=mesh,
        compiler_params=pltpu.CompilerParams(needs_layout_passes=False,
                                             use_tc_tiling_on_sc=False),
        scratch_types=[
            pltpu.VMEM((_NSYM, _DSP), jnp.float32),
            pltpu.VMEM((_NTIME, _DTP), jnp.float32),
            pltpu.VMEM((_T * _F,), jnp.float32),
            pltpu.VMEM((_T * _F,), jnp.float32),
            pltpu.VMEM((_T, _DOP), jnp.float32),
            pltpu.VMEM((_T, _DOP), jnp.float32),
            [pltpu.SemaphoreType.DMA, pltpu.SemaphoreType.DMA],
            [pltpu.SemaphoreType.DMA, pltpu.SemaphoreType.DMA],
            pltpu.SemaphoreType.DMA,
        ],
    )(xf, embs, embt)


def kernel(x, emb_s, emb_t):
    xf = x.reshape(_N_TOK * _F)
    out = _sc_embed(xf, emb_s, emb_t)
    return out.reshape(_B, _L, _DO)


# speedup vs baseline: 4.2776x; 1.0705x over previous
import jax
import jax.numpy as jnp
from jax import lax
from jax.experimental import pallas as pl
from jax.experimental.pallas import tpu as pltpu
from jax.experimental.pallas import tpu_sc as plsc

_B, _L, _F = 16384, 50, 10
_DS, _DT = 32, 32
_NSYM, _NTIME = 100, 2000
_DO = _F - 2 + _DS + _DT
_DOP = _DO + 1
_DSP, _DTP = _DS + 1, _DT + 1

_NC, _NS, _LANES = 2, 16, 16
_NW = _NC * _NS
_N_TOK = _B * _L
_TPW = _N_TOK // _NW
_T = 256
_NCHUNK = _TPW // _T
_G = _T // _LANES


def _body(x_hbm, embs_hbm, embt_hbm, out_hbm,
          embs_v, embt_v, xv0, xv1, ov0, ov1, sin, sout, stab):
    wid = lax.axis_index("s") * _NC + lax.axis_index("c")
    base = wid * _TPW
    xvs = (xv0, xv1)
    ovs = (ov0, ov1)

    pltpu.async_copy(embs_hbm, embs_v.at[:, pl.ds(0, _DS)], stab).wait()
    pltpu.async_copy(embt_hbm, embt_v.at[:, pl.ds(0, _DT)], stab).wait()

    iota = lax.iota(jnp.int32, _LANES)

    def start_in(i, b):
        t0 = base + i * _T
        return pltpu.async_copy(x_hbm.at[pl.ds(t0 * _F, _T * _F)], xvs[b],
                                sin[b])

    def wait_in(b):
        pltpu.make_async_copy(x_hbm.at[pl.ds(0, _T * _F)], xvs[b],
                              sin[b]).wait()

    def start_out(i, b):
        t0 = base + i * _T
        return pltpu.async_copy(ovs[b].at[:, pl.ds(0, _DO)],
                                out_hbm.at[pl.ds(t0, _T)], sout[b])

    def wait_out(b):
        pltpu.make_async_copy(out_hbm.at[pl.ds(0, _T)],
                              ovs[b].at[:, pl.ds(0, _DO)], sout[b]).wait()

    def compute(b):
        xv, ov = xvs[b], ovs[b]

        @plsc.parallel_loop(0, _G, step=1, unroll=1)
        def group(g):
            tv = iota + g * _LANES
            xb = tv * _F
            srow = plsc.load_gather(xv, [xb + (_F - 2)]).astype(jnp.int32)
            trow = plsc.load_gather(xv, [xb + (_F - 1)]).astype(jnp.int32)
            for c in range(_F - 2):
                v = plsc.load_gather(xv, [xb + c])
                plsc.store_scatter(ov, [tv, jnp.full((_LANES,), c,
                                                     jnp.int32)], v)
            for d in range(_DS):
                v = plsc.load_gather(embs_v, [srow,
                                              jnp.full((_LANES,), d, jnp.int32)])
                plsc.store_scatter(ov, [tv, jnp.full((_LANES,), _F - 2 + d,
                                                     jnp.int32)], v)
            for d in range(_DT):
                v = plsc.load_gather(embt_v, [trow,
                                              jnp.full((_LANES,), d, jnp.int32)])
                plsc.store_scatter(ov,
                                   [tv, jnp.full((_LANES,), _F - 2 + _DS + d,
                                                 jnp.int32)], v)

    start_in(0, 0)
    start_in(1, 1)

    def pair(j, carry):
        i0 = j * 2
        for b in range(2):
            i = i0 + b
            @pl.when(i >= 2)
            def _():
                wait_out(b)
            wait_in(b)
            compute(b)
            start_out(i, b)

            @pl.when(i + 2 < _NCHUNK)
            def _():
                start_in(i + 2, b)
        return carry

    lax.fori_loop(0, _NCHUNK // 2, pair, 0)
    wait_out(0)
    wait_out(1)


@jax.jit
def _sc_embed(xf, embs, embt):
    mesh = plsc.VectorSubcoreMesh(core_axis_name="c", subcore_axis_name="s")
    return pl.kernel(
        _body,
        out_type=jax.ShapeDtypeStruct((_N_TOK, _DO), jnp.float32),
        mesh=mesh,
        compiler_params=pltpu.CompilerParams(needs_layout_passes=False,
                                             use_tc_tiling_on_sc=False),
        scratch_types=[
            pltpu.VMEM((_NSYM, _DSP), jnp.float32),
            pltpu.VMEM((_NTIME, _DTP), jnp.float32),
            pltpu.VMEM((_T * _F,), jnp.float32),
            pltpu.VMEM((_T * _F,), jnp.float32),
            pltpu.VMEM((_T, _DOP), jnp.float32),
            pltpu.VMEM((_T, _DOP), jnp.float32),
            [pltpu.SemaphoreType.DMA, pltpu.SemaphoreType.DMA],
            [pltpu.SemaphoreType.DMA, pltpu.SemaphoreType.DMA],
            pltpu.SemaphoreType.DMA,
        ],
    )(xf, embs, embt)


def kernel(x, emb_s, emb_t):
    xf = x.reshape(_N_TOK * _F)
    out = _sc_embed(xf, emb_s, emb_t)
    return out.reshape(_B, _L, _DO)


# speedup vs baseline: 11.2824x; 2.6375x over previous
import jax
import jax.numpy as jnp
from jax import lax
from jax.experimental import pallas as pl
from jax.experimental.pallas import tpu as pltpu
from jax.experimental.pallas import tpu_sc as plsc

_B, _L, _F = 16384, 50, 10
_DS, _DT = 32, 32
_NSYM, _NTIME = 100, 2000
_DO = _F - 2 + _DS + _DT
_CT = _DO // 8
_DSP, _DTP = _DS + 1, _DT + 1

_NC, _NS, _LANES = 2, 16, 16
_NW = _NC * _NS
_BT = _B // 128
_NUNIT = _L * _BT
_UPW = _NUNIT // _NW
_G = 128 // _LANES


def _body(xt_hbm, embs_hbm, embt_hbm, out_hbm,
          embs_v, embt_v, xv0, xv1, ov0, ov1, sin, sout, stab):
    wid = lax.axis_index("s") * _NC + lax.axis_index("c")
    base = wid * _UPW
    xvs = (xv0, xv1)
    ovs = (ov0, ov1)

    pltpu.async_copy(embs_hbm, embs_v.at[:, pl.ds(0, _DS)], stab).wait()
    pltpu.async_copy(embt_hbm, embt_v.at[:, pl.ds(0, _DT)], stab).wait()

    def start_in(u, b):
        l, bt = u // _BT, u % _BT
        return pltpu.async_copy(xt_hbm.at[:, l, pl.ds(bt * 128, 128)],
                                xvs[b], sin[b])

    def wait_in(b):
        pltpu.make_async_copy(xt_hbm.at[:, 0, pl.ds(0, 128)], xvs[b],
                              sin[b]).wait()

    def start_out(u, b):
        l, bt = u // _BT, u % _BT
        return pltpu.async_copy(ovs[b], out_hbm.at[l, :, bt], sout[b])

    def wait_out(b):
        pltpu.make_async_copy(out_hbm.at[0, :, 0], ovs[b], sout[b]).wait()

    def compute(b):
        xv, ov = xvs[b], ovs[b]
        for g in range(_G):
            j0 = g * _LANES
            sidx = xv[_F - 2, pl.ds(j0, _LANES)].astype(jnp.int32)
            tidx = xv[_F - 1, pl.ds(j0, _LANES)].astype(jnp.int32)
            for c in range(_F - 2):
                ov[c // 8, c % 8, pl.ds(j0, _LANES)] = xv[c, pl.ds(j0, _LANES)]
            for d in range(_DS):
                v = plsc.load_gather(
                    embs_v, [sidx, jnp.full((_LANES,), d, jnp.int32)])
                c = _F - 2 + d
                ov[c // 8, c % 8, pl.ds(j0, _LANES)] = v
            for d in range(_DT):
                v = plsc.load_gather(
                    embt_v, [tidx, jnp.full((_LANES,), d, jnp.int32)])
                c = _F - 2 + _DS + d
                ov[c // 8, c % 8, pl.ds(j0, _LANES)] = v

    start_in(base, 0)
    start_in(base + 1, 1)

    def pair(j, carry):
        u0 = base + j * 2
        for b in range(2):
            u = u0 + b
            @pl.when(j * 2 + b >= 2)
            def _():
                wait_out(b)
            wait_in(b)
            compute(b)
            start_out(u, b)

            @pl.when(j * 2 + b + 2 < _UPW)
            def _():
                start_in(u + 2, b)
        return carry

    lax.fori_loop(0, _UPW // 2, pair, 0)
    wait_out(0)
    wait_out(1)


@jax.jit
def _sc_embed(xt, embs, embt):
    mesh = plsc.VectorSubcoreMesh(core_axis_name="c", subcore_axis_name="s")
    return pl.kernel(
        _body,
        out_type=jax.ShapeDtypeStruct((_L, _CT, _BT, 8, 128), jnp.float32),
        mesh=mesh,
        compiler_params=pltpu.CompilerParams(needs_layout_passes=False,
                                             use_tc_tiling_on_sc=False),
        scratch_types=[
            pltpu.VMEM((_NSYM, _DSP), jnp.float32),
            pltpu.VMEM((_NTIME, _DTP), jnp.float32),
            pltpu.VMEM((_F, 128), jnp.float32),
            pltpu.VMEM((_F, 128), jnp.float32),
            pltpu.VMEM((_CT, 8, 128), jnp.float32),
            pltpu.VMEM((_CT, 8, 128), jnp.float32),
            [pltpu.SemaphoreType.DMA, pltpu.SemaphoreType.DMA],
            [pltpu.SemaphoreType.DMA, pltpu.SemaphoreType.DMA],
            pltpu.SemaphoreType.DMA,
        ],
    )(xt, embs, embt)


def kernel(x, emb_s, emb_t):
    xt = jnp.transpose(x, (2, 1, 0))
    out5 = _sc_embed(xt, emb_s, emb_t)
    return jnp.transpose(out5, (2, 4, 0, 1, 3)).reshape(_B, _L, _DO)
